# baseline (speedup 1.0000x reference)
import jax
import jax.numpy as jnp
from jax.experimental import pallas as pl
from jax.experimental.pallas import tpu as pltpu


def _d_from_rowsum(rs):
    d = jax.lax.rsqrt(rs)
    return jnp.where(rs > 0.0, d, jnp.where(rs == 0.0, 0.0, d))


def _read_w(w_ref):
    T = w_ref.shape[0]
    return [w_ref[t, 0] for t in range(T)]


def _pass1_kernel(w_ref, sup_ref, x_ref, w0_ref, rs_out, a0_out,
                  xx_scr, rs_scr):
    i = pl.program_id(0)
    ng = pl.num_programs(0)
    ws = _read_w(w_ref)
    T = len(ws)
    blk = sup_ref[...]
    BM, K = blk.shape

    lane = jax.lax.broadcasted_iota(jnp.int32, (1, K), 1) % T
    wrow = jnp.zeros((1, K), jnp.float32)
    for t in range(T):
        wrow = jnp.where(lane == t, ws[t], wrow)
    rs_blk = jnp.sum(blk * wrow, axis=1, keepdims=True)
    rs_scr[pl.ds(i * BM, BM), :] = rs_blk

    xb = x_ref[...]
    KX = xb.shape[1]
    IN = KX // T
    r2 = jax.lax.broadcasted_iota(jnp.int32, (KX, IN), 0)
    c2 = jax.lax.broadcasted_iota(jnp.int32, (KX, IN), 1)
    wsel = jnp.zeros((KX, IN), jnp.float32)
    for t in range(T):
        wsel = jnp.where(r2 % T == t, ws[t], wsel)
    px = jnp.where(r2 // T == c2, wsel, 0.0) / T
    xx_scr[pl.ds(i * BM, BM), :] = jnp.dot(
        xb, px, preferred_element_type=jnp.float32)

    @pl.when(i == ng - 1)
    def _():
        rs_out[...] = rs_scr[...]
        d_all = _d_from_rowsum(rs_scr[...])
        a0_out[...] = jnp.dot(xx_scr[...], w0_ref[...],
                              preferred_element_type=jnp.float32) * d_all


def _pass2_kernel(sup_ref, b0_ref, rs_ref, w1_ref, a1_out, h1_scr):
    i = pl.program_id(0)
    ng = pl.num_programs(0)
    BM = sup_ref.shape[0]
    g = jnp.dot(sup_ref[...], b0_ref[...],
                preferred_element_type=jnp.float32)
    d_blk = _d_from_rowsum(rs_ref[pl.ds(i * BM, BM), :])
    g = g * d_blk
    h1_scr[pl.ds(i * BM, BM), :] = jnp.where(g >= 0.0, g, 0.01 * g)

    @pl.when(i == ng - 1)
    def _():
        d_all = _d_from_rowsum(rs_ref[...])
        a1_out[...] = jnp.dot(h1_scr[...], w1_ref[...],
                              preferred_element_type=jnp.float32) * d_all


def _pass3_kernel(sup_ref, b1_ref, rs_ref, tva_ref, wtb_ref,
                  prob_out, tv_out):
    i = pl.program_id(0)
    BM = sup_ref.shape[0]
    g = jnp.dot(sup_ref[...], b1_ref[...],
                preferred_element_type=jnp.float32)
    d_blk = _d_from_rowsum(rs_ref[pl.ds(i * BM, BM), :])
    g = g * d_blk
    h = jnp.where(g >= 0.0, g, 0.01 * g)
    m = jnp.max(h, axis=1, keepdims=True)
    e = jnp.exp(h - m)
    prob_out[...] = e / jnp.sum(e, axis=1, keepdims=True)
    t = jnp.dot(tva_ref[...], wtb_ref[...],
                preferred_element_type=jnp.float32)
    tv_out[...] = jnp.maximum(t, 0.0)


def kernel(x, support, time_vector, w_adj_weight, W0, W1, Wt, bt):
    N, IN_DIM, T = x.shape
    H0 = W0.shape[1]
    H1 = W1.shape[1]
    OUT = Wt.shape[0]
    f32 = jnp.float32

    sup_r = support.reshape(N, N * T)
    x_r = x.reshape(N, IN_DIM * T)
    wv = w_adj_weight.reshape(T)

    BM = 256
    G = N // BM

    rowsum, A0 = pl.pallas_call(
        _pass1_kernel,
        grid=(G,),
        in_specs=[
            pl.BlockSpec(memory_space=pltpu.SMEM),
            pl.BlockSpec((BM, N * T), lambda i: (i, 0)),
            pl.BlockSpec((BM, IN_DIM * T), lambda i: (i, 0)),
            pl.BlockSpec((IN_DIM, H0), lambda i: (0, 0)),
        ],
        out_specs=[
            pl.BlockSpec((N, 1), lambda i: (0, 0)),
            pl.BlockSpec((N, H0), lambda i: (0, 0)),
        ],
        out_shape=[
            jax.ShapeDtypeStruct((N, 1), f32),
            jax.ShapeDtypeStruct((N, H0), f32),
        ],
        scratch_shapes=[pltpu.VMEM((N, IN_DIM), f32), pltpu.VMEM((N, 1), f32)],
    )(w_adj_weight, sup_r, x_r, W0)

    def expand(a):
        return (a[:, None, :] * wv[None, :, None]).reshape(N * T, a.shape[1])

    A1 = pl.pallas_call(
        _pass2_kernel,
        grid=(G,),
        in_specs=[
            pl.BlockSpec((BM, N * T), lambda i: (i, 0)),
            pl.BlockSpec((N * T, H0), lambda i: (0, 0)),
            pl.BlockSpec((N, 1), lambda i: (0, 0)),
            pl.BlockSpec((H0, H1), lambda i: (0, 0)),
        ],
        out_specs=pl.BlockSpec((N, H1), lambda i: (0, 0)),
        out_shape=jax.ShapeDtypeStruct((N, H1), f32),
        scratch_shapes=[pltpu.VMEM((N, H0), f32)],
    )(sup_r, expand(A0), rowsum, W1)

    tva = jnp.concatenate([time_vector, jnp.ones((N, 1), f32)], axis=1)
    wtb = jnp.concatenate([Wt.T, bt[None, :]], axis=0)

    prob, tv = pl.pallas_call(
        _pass3_kernel,
        grid=(G,),
        in_specs=[
            pl.BlockSpec((BM, N * T), lambda i: (i, 0)),
            pl.BlockSpec((N * T, H1), lambda i: (0, 0)),
            pl.BlockSpec((N, 1), lambda i: (0, 0)),
            pl.BlockSpec((BM, T + 1), lambda i: (i, 0)),
            pl.BlockSpec((T + 1, OUT), lambda i: (0, 0)),
        ],
        out_specs=[
            pl.BlockSpec((BM, H1), lambda i: (i, 0)),
            pl.BlockSpec((BM, OUT), lambda i: (i, 0)),
        ],
        out_shape=[
            jax.ShapeDtypeStruct((N, H1), f32),
            jax.ShapeDtypeStruct((N, OUT), f32),
        ],
    )(sup_r, expand(A1), rowsum, tva, wtb)

    return (prob, tv)


# speedup vs baseline: 1.1309x; 1.1309x over previous
import jax
import jax.numpy as jnp
from jax.experimental import pallas as pl
from jax.experimental.pallas import tpu as pltpu


def _d_from_rowsum(rs):
    d = jax.lax.rsqrt(rs)
    return jnp.where(rs > 0.0, d, jnp.where(rs == 0.0, 0.0, d))


def _leaky(g):
    return jnp.where(g >= 0.0, g, 0.01 * g)


def _fold_matrix(ws, rows, cols, scale):
    T = len(ws)
    r2 = jax.lax.broadcasted_iota(jnp.int32, (rows, cols), 0)
    c2 = jax.lax.broadcasted_iota(jnp.int32, (rows, cols), 1)
    wsel = jnp.zeros((rows, cols), jnp.float32)
    for t in range(T):
        wsel = jnp.where(r2 % T == t, ws[t] * scale, wsel)
    return jnp.where(r2 // T == c2, wsel, 0.0)


def _mono_kernel(w_ref, sup_ref, x_ref, w0_ref, w1_ref, tva_ref, wtb_ref,
                 prob_out, tv_out, s_scr, rs_scr, xx_scr, h1_scr, a1_scr):
    i = pl.program_id(0)
    NB = (pl.num_programs(0) - 1) // 2
    BM = sup_ref.shape[0]
    N = s_scr.shape[0]
    T = w_ref.shape[0]
    CH = 128
    K = T * CH

    @pl.when(i < NB)
    def _collapse():
        ws = [w_ref[t, 0] for t in range(T)]
        pmat = _fold_matrix(ws, K, CH, 1.0).astype(jnp.bfloat16)
        rs_acc = jnp.zeros((BM, 1), jnp.float32)
        for c in range(N // CH):
            chunk = sup_ref[:, c * K:(c + 1) * K].astype(jnp.bfloat16)
            s_c = jnp.dot(chunk, pmat, preferred_element_type=jnp.float32)
            rs_acc = rs_acc + jnp.sum(s_c, axis=1, keepdims=True)
            s_scr[pl.ds(i * BM, BM), pl.ds(c * CH, CH)] = (
                s_c.astype(jnp.bfloat16))
        rs_scr[pl.ds(i * BM, BM), :] = rs_acc

        xb = x_ref[...]
        KX = xb.shape[1]
        px = _fold_matrix(ws, KX, KX // T, 1.0 / T)
        xx_scr[pl.ds(i * BM, BM), :] = jnp.dot(
            xb, px, preferred_element_type=jnp.float32)

    @pl.when(i == NB)
    def _layer1():
        d_all = _d_from_rowsum(rs_scr[...])
        a0 = (jnp.dot(xx_scr[...], w0_ref[...],
                      preferred_element_type=jnp.float32)
              * d_all).astype(jnp.bfloat16)
        for b in range(N // BM):
            g = jnp.dot(s_scr[pl.ds(b * BM, BM), :], a0,
                        preferred_element_type=jnp.float32)
            g = g * _d_from_rowsum(rs_scr[pl.ds(b * BM, BM), :])
            h1_scr[pl.ds(b * BM, BM), :] = _leaky(g)
        a1_scr[...] = (jnp.dot(h1_scr[...], w1_ref[...],
                               preferred_element_type=jnp.float32)
                       * d_all).astype(jnp.bfloat16)

    @pl.when(i > NB)
    def _layer2():
        b = i - NB - 1
        g = jnp.dot(s_scr[pl.ds(b * BM, BM), :], a1_scr[...],
                    preferred_element_type=jnp.float32)
        g = g * _d_from_rowsum(rs_scr[pl.ds(b * BM, BM), :])
        h = _leaky(g)
        m = jnp.max(h, axis=1, keepdims=True)
        e = jnp.exp(h - m)
        prob_out[...] = e / jnp.sum(e, axis=1, keepdims=True)
        t = jnp.dot(tva_ref[...], wtb_ref[...],
                    preferred_element_type=jnp.float32)
        tv_out[...] = jnp.maximum(t, 0.0)


def kernel(x, support, time_vector, w_adj_weight, W0, W1, Wt, bt):
    N, IN_DIM, T = x.shape
    H0 = W0.shape[1]
    H1 = W1.shape[1]
    OUT = Wt.shape[0]
    f32 = jnp.float32

    sup_r = support.reshape(N, N * T)
    x_r = x.reshape(N, IN_DIM * T)

    BM = 64
    NB = N // BM
    last = NB - 1

    tva = jnp.concatenate([time_vector, jnp.ones((N, 1), f32)], axis=1)
    wtb = jnp.concatenate([Wt.T, bt[None, :]], axis=0)

    def out_map(i):
        return (jnp.clip(i - NB - 1, 0, last), 0)

    prob, tv = pl.pallas_call(
        _mono_kernel,
        grid=(2 * NB + 1,),
        in_specs=[
            pl.BlockSpec(memory_space=pltpu.SMEM),
            pl.BlockSpec((BM, N * T), lambda i: (jnp.minimum(i, last), 0)),
            pl.BlockSpec((BM, IN_DIM * T),
                         lambda i: (jnp.minimum(i, last), 0)),
            pl.BlockSpec((IN_DIM, H0), lambda i: (0, 0)),
            pl.BlockSpec((H0, H1), lambda i: (0, 0)),
            pl.BlockSpec((BM, T + 1), out_map),
            pl.BlockSpec((T + 1, OUT), lambda i: (0, 0)),
        ],
        out_specs=[
            pl.BlockSpec((BM, H1), out_map),
            pl.BlockSpec((BM, OUT), out_map),
        ],
        out_shape=[
            jax.ShapeDtypeStruct((N, H1), f32),
            jax.ShapeDtypeStruct((N, OUT), f32),
        ],
        scratch_shapes=[
            pltpu.VMEM((N, N), jnp.bfloat16),
            pltpu.VMEM((N, 1), f32),
            pltpu.VMEM((N, IN_DIM), f32),
            pltpu.VMEM((N, H0), f32),
            pltpu.VMEM((N, H1), jnp.bfloat16),
        ],
    )(w_adj_weight, sup_r, x_r, W0, W1, tva, wtb)

    return (prob, tv)


# speedup vs baseline: 1.4112x; 1.2479x over previous
import jax
import jax.numpy as jnp
from jax.experimental import pallas as pl
from jax.experimental.pallas import tpu as pltpu


def _d_from_rowsum(rs):
    d = jax.lax.rsqrt(rs)
    return jnp.where(rs > 0.0, d, jnp.where(rs == 0.0, 0.0, d))


def _leaky(g):
    return jnp.where(g >= 0.0, g, 0.01 * g)


def _fold_matrix(ws, rows, cols, scale):
    T = len(ws)
    r2 = jax.lax.broadcasted_iota(jnp.int32, (rows, cols), 0)
    c2 = jax.lax.broadcasted_iota(jnp.int32, (rows, cols), 1)
    wsel = jnp.zeros((rows, cols), jnp.float32)
    for t in range(T):
        wsel = jnp.where(r2 % T == t, ws[t] * scale, wsel)
    return jnp.where(r2 // T == c2, wsel, 0.0)


def _mono_kernel(w_ref, sup_ref, x_ref, w0_ref, w1_ref, tva_ref, wtb_ref,
                 prob_out, tv_out, s_scr, rs_scr, xx_scr, h1_scr, a1_scr):
    i = pl.program_id(0)
    NB = (pl.num_programs(0) - 1) // 2
    BM = sup_ref.shape[0]
    N = s_scr.shape[0]
    T = w_ref.shape[0]
    CH = 128
    K = T * CH

    @pl.when(i < NB)
    def _collapse():
        ws = [w_ref[t, 0] for t in range(T)]
        pmat = _fold_matrix(ws, K, CH, 1.0).astype(jnp.bfloat16)
        rs_acc = jnp.zeros((BM, 1), jnp.float32)
        for c in range(N // CH):
            chunk = sup_ref[:, c * K:(c + 1) * K]
            s_c = jnp.dot(chunk, pmat, preferred_element_type=jnp.float32)
            rs_acc = rs_acc + jnp.sum(s_c, axis=1, keepdims=True)
            s_scr[pl.ds(i * BM, BM), pl.ds(c * CH, CH)] = (
                s_c.astype(jnp.bfloat16))
        rs_scr[pl.ds(i * BM, BM), :] = rs_acc

        xb = x_ref[...]
        KX = xb.shape[1]
        px = _fold_matrix(ws, KX, KX // T, 1.0 / T)
        xx_scr[pl.ds(i * BM, BM), :] = jnp.dot(
            xb, px, preferred_element_type=jnp.float32)

    @pl.when(i == NB)
    def _layer1():
        d_all = _d_from_rowsum(rs_scr[...])
        a0 = (jnp.dot(xx_scr[...], w0_ref[...],
                      preferred_element_type=jnp.float32)
              * d_all).astype(jnp.bfloat16)
        for b in range(N // BM):
            g = jnp.dot(s_scr[pl.ds(b * BM, BM), :], a0,
                        preferred_element_type=jnp.float32)
            g = g * _d_from_rowsum(rs_scr[pl.ds(b * BM, BM), :])
            h1_scr[pl.ds(b * BM, BM), :] = _leaky(g)
        a1_scr[...] = (jnp.dot(h1_scr[...], w1_ref[...],
                               preferred_element_type=jnp.float32)
                       * d_all).astype(jnp.bfloat16)

    @pl.when(i > NB)
    def _layer2():
        b = i - NB - 1
        g = jnp.dot(s_scr[pl.ds(b * BM, BM), :], a1_scr[...],
                    preferred_element_type=jnp.float32)
        g = g * _d_from_rowsum(rs_scr[pl.ds(b * BM, BM), :])
        h = _leaky(g)
        m = jnp.max(h, axis=1, keepdims=True)
        e = jnp.exp(h - m)
        prob_out[...] = e / jnp.sum(e, axis=1, keepdims=True)
        t = jnp.dot(tva_ref[...], wtb_ref[...],
                    preferred_element_type=jnp.float32)
        tv_out[...] = jnp.maximum(t, 0.0)


def kernel(x, support, time_vector, w_adj_weight, W0, W1, Wt, bt):
    N, IN_DIM, T = x.shape
    H0 = W0.shape[1]
    H1 = W1.shape[1]
    OUT = Wt.shape[0]
    f32 = jnp.float32

    sup_r = support.astype(jnp.bfloat16).reshape(N, N * T)
    x_r = x.reshape(N, IN_DIM * T)

    BM = 128
    NB = N // BM
    last = NB - 1

    tva = jnp.concatenate([time_vector, jnp.ones((N, 1), f32)], axis=1)
    wtb = jnp.concatenate([Wt.T, bt[None, :]], axis=0)

    def out_map(i):
        return (jnp.clip(i - NB - 1, 0, last), 0)

    prob, tv = pl.pallas_call(
        _mono_kernel,
        grid=(2 * NB + 1,),
        in_specs=[
            pl.BlockSpec(memory_space=pltpu.SMEM),
            pl.BlockSpec((BM, N * T), lambda i: (jnp.minimum(i, last), 0)),
            pl.BlockSpec((BM, IN_DIM * T),
                         lambda i: (jnp.minimum(i, last), 0)),
            pl.BlockSpec((IN_DIM, H0), lambda i: (0, 0)),
            pl.BlockSpec((H0, H1), lambda i: (0, 0)),
            pl.BlockSpec((BM, T + 1), out_map),
            pl.BlockSpec((T + 1, OUT), lambda i: (0, 0)),
        ],
        out_specs=[
            pl.BlockSpec((BM, H1), out_map),
            pl.BlockSpec((BM, OUT), out_map),
        ],
        out_shape=[
            jax.ShapeDtypeStruct((N, H1), f32),
            jax.ShapeDtypeStruct((N, OUT), f32),
        ],
        scratch_shapes=[
            pltpu.VMEM((N, N), jnp.bfloat16),
            pltpu.VMEM((N, 1), f32),
            pltpu.VMEM((N, IN_DIM), f32),
            pltpu.VMEM((N, H0), f32),
            pltpu.VMEM((N, H1), jnp.bfloat16),
        ],
    )(w_adj_weight, sup_r, x_r, W0, W1, tva, wtb)

    return (prob, tv)
